# baseline (speedup 1.0000x reference)
import functools

import jax
import jax.numpy as jnp
from jax import lax
from jax.experimental import pallas as pl
from jax.experimental.pallas import tpu as pltpu
from jax.experimental.pallas import tpu_sc as plsc

M = 100000
B = 16384
H = 128
I = 32

NC = 2
NS = 16
NW = NC * NS
R = 3128
BPW = B // NW
RPAD = 3136
CWROWS = 26

_mesh = plsc.VectorSubcoreMesh(core_axis_name="c", subcore_axis_name="s")
_sc_params = pltpu.CompilerParams(needs_layout_passes=False)


@functools.partial(
    pl.kernel,
    out_type=jax.ShapeDtypeStruct((B, H), jnp.float32),
    mesh=_mesh,
    compiler_params=_sc_params,
    scratch_types=[
        pltpu.VMEM((BPW,), jnp.int32),
        pltpu.VMEM((BPW, H), jnp.float32),
        pltpu.SemaphoreType.DMA,
    ],
)
def _gather_rows(table_hbm, idx_hbm, out_hbm, idx_v, rows_v, sem):
    wid = lax.axis_index("s") * NC + lax.axis_index("c")
    base = wid * BPW
    pltpu.sync_copy(idx_hbm.at[pl.ds(base, BPW)], idx_v)
    copies = []
    for c in range(BPW // 128):
        copies.append(
            pltpu.async_copy(
                table_hbm.at[idx_v.at[pl.ds(c * 128, 128)]],
                rows_v.at[pl.ds(c * 128, 128)],
                sem,
            )
        )
    for cp in copies:
        cp.wait()
    pltpu.sync_copy(rows_v, out_hbm.at[pl.ds(base, BPW)])


def _gru_body(x_ref, h_ref, wih_ref, whh_ref, bih_ref, bhh_ref, out_ref):
    x = x_ref[...]
    h = h_ref[...]
    dn = (((1,), (1,)), ((), ()))
    gi = lax.dot_general(x, wih_ref[...], dn, preferred_element_type=jnp.float32)
    gi = gi + bih_ref[...]
    gh = lax.dot_general(h, whh_ref[...], dn, preferred_element_type=jnp.float32)
    gh = gh + bhh_ref[...]
    r = jax.nn.sigmoid(gi[:, :H] + gh[:, :H])
    z = jax.nn.sigmoid(gi[:, H:2 * H] + gh[:, H:2 * H])
    n = jnp.tanh(gi[:, 2 * H:] + r * gh[:, 2 * H:])
    out_ref[...] = (1.0 - z) * n + z * h


_TB = 512


def _gru(x, h_g, w_ih, w_hh, b_ih2, b_hh2):
    return pl.pallas_call(
        _gru_body,
        grid=(B // _TB,),
        in_specs=[
            pl.BlockSpec((_TB, I), lambda i: (i, 0)),
            pl.BlockSpec((_TB, H), lambda i: (i, 0)),
            pl.BlockSpec((3 * H, I), lambda i: (0, 0)),
            pl.BlockSpec((3 * H, H), lambda i: (0, 0)),
            pl.BlockSpec((1, 3 * H), lambda i: (0, 0)),
            pl.BlockSpec((1, 3 * H), lambda i: (0, 0)),
        ],
        out_specs=pl.BlockSpec((_TB, H), lambda i: (i, 0)),
        out_shape=jax.ShapeDtypeStruct((B, H), jnp.float32),
    )(x, h_g, w_ih, w_hh, b_ih2, b_hh2)


@functools.partial(
    pl.kernel,
    mesh=_mesh,
    compiler_params=_sc_params,
    scratch_types=[
        pltpu.VMEM((B,), jnp.int32),
        pltpu.VMEM((RPAD,), jnp.int32),
        pltpu.VMEM((CWROWS, 128), jnp.int32),
        pltpu.VMEM((CWROWS, 128), jnp.int32),
        pltpu.VMEM((128, H), jnp.float32),
        pltpu.SemaphoreType.DMA,
    ],
)
def _scatter_update(idx_hbm, hnew_hbm, out_hbm,
                    idx_v, winner, cw_s, cw_b, rowbuf, sem):
    wid = lax.axis_index("s") * NC + lax.axis_index("c")
    lo = wid * R
    nrows = jnp.minimum(jnp.int32(R), jnp.int32(M) - lo)
    lane = lax.iota(jnp.int32, 16)

    pltpu.sync_copy(idx_hbm, idx_v)

    def init_chunk(k, _):
        winner[pl.ds(k * 16, 16)] = jnp.full((16,), -1, jnp.int32)
        return 0

    lax.fori_loop(0, RPAD // 16, init_chunk, 0)

    SENT = jnp.int32(1 << 20)

    def scan_chunk(j, _):
        idx = idx_v[pl.ds(j * 16, 16)]
        il = idx - lo
        owned = (il >= 0) & (il < nrows)

        @pl.when(jnp.any(owned))
        def _():
            key = jnp.where(owned, il * 16 + lane, SENT + lane)
            bvec = j * 16 + lane
            ks, vs = plsc.sort_key_val(key, bvec)
            ils = lax.shift_right_arithmetic(ks, 4)
            nxt = ils.at[jnp.minimum(lane + 1, 15)].get(mode="promise_in_bounds")
            own_s = ks < SENT
            is_last = own_s & ((lane == 15) | (ils != nxt))
            addr = jnp.where(own_s, ils, R)
            plsc.store_scatter(winner, [addr], vs, mask=is_last)

        return 0

    lax.fori_loop(0, B // 16, scan_chunk, 0)

    def compact_chunk(k, carry):
        off, menc = carry
        wv = winner[pl.ds(k * 16, 16)]
        m = wv >= 0
        mi = m.astype(jnp.int32)
        slots = lo + k * 16 + lane
        pos = off + lax.cumsum(mi, axis=0) - 1
        pos = jnp.where(m, pos, CWROWS * 128 - 1)
        prow = lax.shift_right_arithmetic(pos, 7)
        pcol = lax.bitwise_and(pos, jnp.int32(127))
        plsc.store_scatter(cw_s, [prow, pcol], slots, mask=m)
        plsc.store_scatter(cw_b, [prow, pcol], wv, mask=m)
        enc = jnp.where(m, slots * 16384 + wv, -1)
        return off + jnp.sum(mi), jnp.maximum(menc, jnp.max(enc))

    w_t, menc = lax.fori_loop(
        0, RPAD // 16, compact_chunk, (jnp.int32(0), jnp.int32(-1))
    )

    @pl.when(w_t > 0)
    def _():
        s_last = lax.shift_right_arithmetic(menc, 14)
        b_last = lax.bitwise_and(menc, jnp.int32(16383))
        target = ((w_t + 127) // 128) * 128

        def pad_chunk(k, _):
            addr = w_t + k * 16 + lane
            mk = addr < target
            addr = jnp.minimum(addr, CWROWS * 128 - 1)
            prow = lax.shift_right_arithmetic(addr, 7)
            pcol = lax.bitwise_and(addr, jnp.int32(127))
            plsc.store_scatter(cw_s, [prow, pcol], jnp.full((16,), s_last), mask=mk)
            plsc.store_scatter(cw_b, [prow, pcol], jnp.full((16,), b_last), mask=mk)
            return 0

        lax.fori_loop(0, 8, pad_chunk, 0)
        nch = target // 128

        def scatter_chunk(c, _):
            pltpu.async_copy(hnew_hbm.at[cw_b.at[c]], rowbuf, sem).wait()
            pltpu.async_copy(rowbuf, out_hbm.at[cw_s.at[c]], sem).wait()
            return 0

        lax.fori_loop(0, nch, scatter_chunk, 0)


def kernel(current_time, mgn_h, delta_t, X_obs, i_obs, update, W_ih, W_hh, b_ih, b_hh):
    idx = i_obs.astype(jnp.int32)
    h_g = _gather_rows(mgn_h, idx)
    h_new = _gru(X_obs, h_g, W_ih, W_hh,
                 b_ih.reshape(1, 3 * H), b_hh.reshape(1, 3 * H))
    out_ref = jax.new_ref(mgn_h)
    _scatter_update(idx, h_new, out_ref)
    return out_ref[...]


# speedup vs baseline: 1.1094x; 1.1094x over previous
import functools

import jax
import jax.numpy as jnp
from jax import lax
from jax.experimental import pallas as pl
from jax.experimental.pallas import tpu as pltpu
from jax.experimental.pallas import tpu_sc as plsc

M = 100000
B = 16384
H = 128
I = 32

NC = 2
NS = 16
NW = NC * NS
R = 3128
BPW = B // NW
RPAD = 3136
CWROWS = 26

_mesh = plsc.VectorSubcoreMesh(core_axis_name="c", subcore_axis_name="s")
_sc_params = pltpu.CompilerParams(needs_layout_passes=False)


@functools.partial(
    pl.kernel,
    out_type=jax.ShapeDtypeStruct((B, H), jnp.float32),
    mesh=_mesh,
    compiler_params=_sc_params,
    scratch_types=[
        pltpu.VMEM((BPW,), jnp.int32),
        pltpu.VMEM((BPW, H), jnp.float32),
        pltpu.SemaphoreType.DMA,
    ],
)
def _gather_rows(table_hbm, idx_hbm, out_hbm, idx_v, rows_v, sem):
    wid = lax.axis_index("s") * NC + lax.axis_index("c")
    base = wid * BPW
    pltpu.sync_copy(idx_hbm.at[pl.ds(base, BPW)], idx_v)
    copies = []
    for c in range(BPW // 128):
        copies.append(
            pltpu.async_copy(
                table_hbm.at[idx_v.at[pl.ds(c * 128, 128)]],
                rows_v.at[pl.ds(c * 128, 128)],
                sem,
            )
        )
    for cp in copies:
        cp.wait()
    pltpu.sync_copy(rows_v, out_hbm.at[pl.ds(base, BPW)])


def _gru_body(x_ref, h_ref, wih_ref, whh_ref, bih_ref, bhh_ref, out_ref):
    x = x_ref[...]
    h = h_ref[...]
    dn = (((1,), (1,)), ((), ()))
    gi = lax.dot_general(x, wih_ref[...], dn, preferred_element_type=jnp.float32)
    gi = gi + bih_ref[...]
    gh = lax.dot_general(h, whh_ref[...], dn, preferred_element_type=jnp.float32)
    gh = gh + bhh_ref[...]
    r = jax.nn.sigmoid(gi[:, :H] + gh[:, :H])
    z = jax.nn.sigmoid(gi[:, H:2 * H] + gh[:, H:2 * H])
    n = jnp.tanh(gi[:, 2 * H:] + r * gh[:, 2 * H:])
    out_ref[...] = (1.0 - z) * n + z * h


_TB = 512


def _gru(x, h_g, w_ih, w_hh, b_ih2, b_hh2):
    return pl.pallas_call(
        _gru_body,
        grid=(B // _TB,),
        in_specs=[
            pl.BlockSpec((_TB, I), lambda i: (i, 0)),
            pl.BlockSpec((_TB, H), lambda i: (i, 0)),
            pl.BlockSpec((3 * H, I), lambda i: (0, 0)),
            pl.BlockSpec((3 * H, H), lambda i: (0, 0)),
            pl.BlockSpec((1, 3 * H), lambda i: (0, 0)),
            pl.BlockSpec((1, 3 * H), lambda i: (0, 0)),
        ],
        out_specs=pl.BlockSpec((_TB, H), lambda i: (i, 0)),
        out_shape=jax.ShapeDtypeStruct((B, H), jnp.float32),
    )(x, h_g, w_ih, w_hh, b_ih2, b_hh2)


@functools.partial(
    pl.kernel,
    mesh=_mesh,
    compiler_params=_sc_params,
    scratch_types=[
        pltpu.VMEM((B,), jnp.int32),
        pltpu.VMEM((RPAD,), jnp.int32),
        pltpu.VMEM((RPAD,), jnp.int32),
        pltpu.VMEM((RPAD,), jnp.int32),
        pltpu.VMEM((RPAD,), jnp.int32),
        pltpu.VMEM((CWROWS, 128), jnp.int32),
        pltpu.VMEM((CWROWS, 128), jnp.int32),
        pltpu.VMEM((128, H), jnp.float32),
        pltpu.SemaphoreType.DMA,
    ],
)
def _scatter_update(idx_hbm, hnew_hbm, out_hbm,
                    idx_v, win0, win1, win2, win3, cw_s, cw_b, rowbuf, sem):
    wid = lax.axis_index("s") * NC + lax.axis_index("c")
    lo = wid * R
    nrows = jnp.minimum(jnp.int32(R), jnp.int32(M) - lo)
    lane = lax.iota(jnp.int32, 16)

    pltpu.sync_copy(idx_hbm, idx_v)

    wins = (win0, win1, win2, win3)

    def init_chunk(k, _):
        neg = jnp.full((16,), -1, jnp.int32)
        for w in wins:
            w[pl.ds(k * 16, 16)] = neg
        return 0

    lax.fori_loop(0, RPAD // 16, init_chunk, 0)

    SENT = jnp.int32(1 << 20)

    def scan_one(j, win):
        idx = idx_v[pl.ds(j * 16, 16)]
        il = idx - lo
        owned = (il >= 0) & (il < nrows)
        key = jnp.where(owned, il * 16 + lane, SENT + lane)
        bvec = j * 16 + lane
        ks, vs = plsc.sort_key_val(key, bvec)
        ils = lax.shift_right_arithmetic(ks, 4)
        nxt = ils.at[jnp.minimum(lane + 1, 15)].get(mode="promise_in_bounds")
        own_s = ks < SENT
        is_last = own_s & ((lane == 15) | (ils != nxt))
        addr = jnp.where(own_s, ils, R)
        plsc.store_scatter(win, [addr], vs, mask=is_last)

    def scan_step(s, _):
        for k in range(4):
            scan_one(s * 4 + k, wins[k])
        return 0

    lax.fori_loop(0, B // 16 // 4, scan_step, 0)

    def compact_chunk(k, carry):
        off, menc = carry
        wv = jnp.maximum(
            jnp.maximum(win0[pl.ds(k * 16, 16)], win1[pl.ds(k * 16, 16)]),
            jnp.maximum(win2[pl.ds(k * 16, 16)], win3[pl.ds(k * 16, 16)]),
        )
        m = wv >= 0
        mi = m.astype(jnp.int32)
        slots = lo + k * 16 + lane
        pos = off + lax.cumsum(mi, axis=0) - 1
        pos = jnp.where(m, pos, CWROWS * 128 - 1)
        prow = lax.shift_right_arithmetic(pos, 7)
        pcol = lax.bitwise_and(pos, jnp.int32(127))
        plsc.store_scatter(cw_s, [prow, pcol], slots, mask=m)
        plsc.store_scatter(cw_b, [prow, pcol], wv, mask=m)
        enc = jnp.where(m, slots * 16384 + wv, -1)
        return off + jnp.sum(mi), jnp.maximum(menc, jnp.max(enc))

    w_t, menc = lax.fori_loop(
        0, RPAD // 16, compact_chunk, (jnp.int32(0), jnp.int32(-1))
    )

    @pl.when(w_t > 0)
    def _():
        s_last = lax.shift_right_arithmetic(menc, 14)
        b_last = lax.bitwise_and(menc, jnp.int32(16383))
        target = ((w_t + 127) // 128) * 128

        def pad_chunk(k, _):
            addr = w_t + k * 16 + lane
            mk = addr < target
            addr = jnp.minimum(addr, CWROWS * 128 - 1)
            prow = lax.shift_right_arithmetic(addr, 7)
            pcol = lax.bitwise_and(addr, jnp.int32(127))
            plsc.store_scatter(cw_s, [prow, pcol], jnp.full((16,), s_last), mask=mk)
            plsc.store_scatter(cw_b, [prow, pcol], jnp.full((16,), b_last), mask=mk)
            return 0

        lax.fori_loop(0, 8, pad_chunk, 0)
        nch = target // 128

        def scatter_chunk(c, _):
            pltpu.async_copy(hnew_hbm.at[cw_b.at[c]], rowbuf, sem).wait()
            pltpu.async_copy(rowbuf, out_hbm.at[cw_s.at[c]], sem).wait()
            return 0

        lax.fori_loop(0, nch, scatter_chunk, 0)


def kernel(current_time, mgn_h, delta_t, X_obs, i_obs, update, W_ih, W_hh, b_ih, b_hh):
    idx = i_obs.astype(jnp.int32)
    h_g = _gather_rows(mgn_h, idx)
    h_new = _gru(X_obs, h_g, W_ih, W_hh,
                 b_ih.reshape(1, 3 * H), b_hh.reshape(1, 3 * H))
    out_ref = jax.new_ref(mgn_h)
    _scatter_update(idx, h_new, out_ref)
    return out_ref[...]


# speedup vs baseline: 1.1593x; 1.0450x over previous
import functools

import jax
import jax.numpy as jnp
from jax import lax
from jax.experimental import pallas as pl
from jax.experimental.pallas import tpu as pltpu
from jax.experimental.pallas import tpu_sc as plsc

M = 100000
B = 16384
H = 128
I = 32

NC = 2
NS = 16
NW = NC * NS
R = 3128
BPW = B // NW
RPAD = 3136
CWROWS = 26

_mesh = plsc.VectorSubcoreMesh(core_axis_name="c", subcore_axis_name="s")
_sc_params = pltpu.CompilerParams(needs_layout_passes=False)


@functools.partial(
    pl.kernel,
    out_type=jax.ShapeDtypeStruct((B, H), jnp.float32),
    mesh=_mesh,
    compiler_params=_sc_params,
    scratch_types=[
        pltpu.VMEM((BPW,), jnp.int32),
        pltpu.VMEM((BPW, H), jnp.float32),
        pltpu.SemaphoreType.DMA,
    ],
)
def _gather_rows(table_hbm, idx_hbm, out_hbm, idx_v, rows_v, sem):
    wid = lax.axis_index("s") * NC + lax.axis_index("c")
    base = wid * BPW
    pltpu.sync_copy(idx_hbm.at[pl.ds(base, BPW)], idx_v)
    copies = []
    for c in range(BPW // 128):
        copies.append(
            pltpu.async_copy(
                table_hbm.at[idx_v.at[pl.ds(c * 128, 128)]],
                rows_v.at[pl.ds(c * 128, 128)],
                sem,
            )
        )
    for cp in copies:
        cp.wait()
    pltpu.sync_copy(rows_v, out_hbm.at[pl.ds(base, BPW)])


def _gru_body(x_ref, h_ref, wih_ref, whh_ref, bih_ref, bhh_ref, tbl_ref,
              out_ref, cp_ref):
    x = x_ref[...]
    h = h_ref[...]
    dn = (((1,), (1,)), ((), ()))
    gi = lax.dot_general(x, wih_ref[...], dn, preferred_element_type=jnp.float32)
    gi = gi + bih_ref[...]
    gh = lax.dot_general(h, whh_ref[...], dn, preferred_element_type=jnp.float32)
    gh = gh + bhh_ref[...]
    r = jax.nn.sigmoid(gi[:, :H] + gh[:, :H])
    z = jax.nn.sigmoid(gi[:, H:2 * H] + gh[:, H:2 * H])
    n = jnp.tanh(gi[:, 2 * H:] + r * gh[:, 2 * H:])
    out_ref[...] = (1.0 - z) * n + z * h
    cp_ref[...] = tbl_ref[...]


_TB = 512


def _gru(x, h_g, w_ih, w_hh, b_ih2, b_hh2, tbl):
    return pl.pallas_call(
        _gru_body,
        grid=(B // _TB,),
        in_specs=[
            pl.BlockSpec((_TB, I), lambda i: (i, 0)),
            pl.BlockSpec((_TB, H), lambda i: (i, 0)),
            pl.BlockSpec((3 * H, I), lambda i: (0, 0)),
            pl.BlockSpec((3 * H, H), lambda i: (0, 0)),
            pl.BlockSpec((1, 3 * H), lambda i: (0, 0)),
            pl.BlockSpec((1, 3 * H), lambda i: (0, 0)),
            pl.BlockSpec((R, H), lambda i: (i, 0)),
        ],
        out_specs=[
            pl.BlockSpec((_TB, H), lambda i: (i, 0)),
            pl.BlockSpec((R, H), lambda i: (i, 0)),
        ],
        out_shape=[
            jax.ShapeDtypeStruct((B, H), jnp.float32),
            jax.ShapeDtypeStruct((M, H), jnp.float32),
        ],
    )(x, h_g, w_ih, w_hh, b_ih2, b_hh2, tbl)


@functools.partial(
    pl.kernel,
    mesh=_mesh,
    compiler_params=_sc_params,
    scratch_types=[
        pltpu.VMEM((B,), jnp.int32),
        pltpu.VMEM((RPAD,), jnp.int32),
        pltpu.VMEM((RPAD,), jnp.int32),
        pltpu.VMEM((RPAD,), jnp.int32),
        pltpu.VMEM((RPAD,), jnp.int32),
        pltpu.VMEM((CWROWS, 128), jnp.int32),
        pltpu.VMEM((CWROWS, 128), jnp.int32),
        pltpu.VMEM((128, H), jnp.float32),
        pltpu.SemaphoreType.DMA,
    ],
)
def _scatter_update(idx_hbm, hnew_hbm, out_hbm,
                    idx_v, win0, win1, win2, win3, cw_s, cw_b, rowbuf, sem):
    wid = lax.axis_index("s") * NC + lax.axis_index("c")
    lo = wid * R
    nrows = jnp.minimum(jnp.int32(R), jnp.int32(M) - lo)
    lane = lax.iota(jnp.int32, 16)

    pltpu.sync_copy(idx_hbm, idx_v)

    wins = (win0, win1, win2, win3)

    def init_chunk(k, _):
        neg = jnp.full((16,), -1, jnp.int32)
        for w in wins:
            w[pl.ds(k * 16, 16)] = neg
        return 0

    lax.fori_loop(0, RPAD // 16, init_chunk, 0)

    SENT = jnp.int32(1 << 20)

    def scan_one(j, win):
        idx = idx_v[pl.ds(j * 16, 16)]
        il = idx - lo
        owned = (il >= 0) & (il < nrows)
        key = jnp.where(owned, il * 16 + lane, SENT + lane)
        bvec = j * 16 + lane
        ks, vs = plsc.sort_key_val(key, bvec)
        ils = lax.shift_right_arithmetic(ks, 4)
        nxt = ils.at[jnp.minimum(lane + 1, 15)].get(mode="promise_in_bounds")
        own_s = ks < SENT
        is_last = own_s & ((lane == 15) | (ils != nxt))
        addr = jnp.where(own_s, ils, R)
        plsc.store_scatter(win, [addr], vs, mask=is_last)

    def scan_step(s, _):
        for k in range(4):
            scan_one(s * 4 + k, wins[k])
        return 0

    lax.fori_loop(0, B // 16 // 4, scan_step, 0)

    def compact_chunk(k, carry):
        off, menc = carry
        wv = jnp.maximum(
            jnp.maximum(win0[pl.ds(k * 16, 16)], win1[pl.ds(k * 16, 16)]),
            jnp.maximum(win2[pl.ds(k * 16, 16)], win3[pl.ds(k * 16, 16)]),
        )
        m = wv >= 0
        mi = m.astype(jnp.int32)
        slots = lo + k * 16 + lane
        pos = off + lax.cumsum(mi, axis=0) - 1
        pos = jnp.where(m, pos, CWROWS * 128 - 1)
        prow = lax.shift_right_arithmetic(pos, 7)
        pcol = lax.bitwise_and(pos, jnp.int32(127))
        plsc.store_scatter(cw_s, [prow, pcol], slots, mask=m)
        plsc.store_scatter(cw_b, [prow, pcol], wv, mask=m)
        enc = jnp.where(m, slots * 16384 + wv, -1)
        return off + jnp.sum(mi), jnp.maximum(menc, jnp.max(enc))

    w_t, menc = lax.fori_loop(
        0, RPAD // 16, compact_chunk, (jnp.int32(0), jnp.int32(-1))
    )

    @pl.when(w_t > 0)
    def _():
        s_last = lax.shift_right_arithmetic(menc, 14)
        b_last = lax.bitwise_and(menc, jnp.int32(16383))
        target = ((w_t + 127) // 128) * 128

        def pad_chunk(k, _):
            addr = w_t + k * 16 + lane
            mk = addr < target
            addr = jnp.minimum(addr, CWROWS * 128 - 1)
            prow = lax.shift_right_arithmetic(addr, 7)
            pcol = lax.bitwise_and(addr, jnp.int32(127))
            plsc.store_scatter(cw_s, [prow, pcol], jnp.full((16,), s_last), mask=mk)
            plsc.store_scatter(cw_b, [prow, pcol], jnp.full((16,), b_last), mask=mk)
            return 0

        lax.fori_loop(0, 8, pad_chunk, 0)
        nch = target // 128

        def scatter_chunk(c, _):
            pltpu.async_copy(hnew_hbm.at[cw_b.at[c]], rowbuf, sem).wait()
            pltpu.async_copy(rowbuf, out_hbm.at[cw_s.at[c]], sem).wait()
            return 0

        lax.fori_loop(0, nch, scatter_chunk, 0)


def kernel(current_time, mgn_h, delta_t, X_obs, i_obs, update, W_ih, W_hh, b_ih, b_hh):
    idx = i_obs.astype(jnp.int32)
    h_g = _gather_rows(mgn_h, idx)
    h_new, tbl_copy = _gru(X_obs, h_g, W_ih, W_hh,
                           b_ih.reshape(1, 3 * H), b_hh.reshape(1, 3 * H),
                           mgn_h)
    out_ref = jax.new_ref(tbl_copy)
    _scatter_update(idx, h_new, out_ref)
    return out_ref[...]


# speedup vs baseline: 1.2170x; 1.0498x over previous
import functools

import jax
import jax.numpy as jnp
from jax import lax
from jax.experimental import pallas as pl
from jax.experimental.pallas import tpu as pltpu
from jax.experimental.pallas import tpu_sc as plsc

M = 100000
B = 16384
H = 128
I = 32

NC = 2
NS = 16
NW = NC * NS
R = 3128
BPW = B // NW
RPAD = 3136
CWROWS = 26

_mesh = plsc.VectorSubcoreMesh(core_axis_name="c", subcore_axis_name="s")
_sc_params = pltpu.CompilerParams(needs_layout_passes=False)


@functools.partial(
    pl.kernel,
    out_type=(
        jax.ShapeDtypeStruct((B, H), jnp.float32),
        jax.ShapeDtypeStruct((NW, CWROWS, 128), jnp.int32),
        jax.ShapeDtypeStruct((NW, CWROWS, 128), jnp.int32),
        jax.ShapeDtypeStruct((NW, 16), jnp.int32),
    ),
    mesh=_mesh,
    compiler_params=_sc_params,
    scratch_types=[
        pltpu.VMEM((B,), jnp.int32),
        pltpu.VMEM((BPW, H), jnp.float32),
        pltpu.VMEM((RPAD,), jnp.int32),
        pltpu.VMEM((RPAD,), jnp.int32),
        pltpu.VMEM((RPAD,), jnp.int32),
        pltpu.VMEM((RPAD,), jnp.int32),
        pltpu.VMEM((CWROWS, 128), jnp.int32),
        pltpu.VMEM((CWROWS, 128), jnp.int32),
        pltpu.VMEM((16,), jnp.int32),
        pltpu.SemaphoreType.DMA,
    ],
)
def _front(table_hbm, idx_hbm, hg_hbm, cws_hbm, cwb_hbm, nch_hbm,
           idx_v, rows_v, win0, win1, win2, win3, cw_s, cw_b, nch_v, sem):
    wid = lax.axis_index("s") * NC + lax.axis_index("c")
    base = wid * BPW
    lo = wid * R
    nrows = jnp.minimum(jnp.int32(R), jnp.int32(M) - lo)
    lane = lax.iota(jnp.int32, 16)

    pltpu.sync_copy(idx_hbm, idx_v)
    copies = []
    for c in range(BPW // 128):
        copies.append(
            pltpu.async_copy(
                table_hbm.at[idx_v.at[pl.ds(base + c * 128, 128)]],
                rows_v.at[pl.ds(c * 128, 128)],
                sem,
            )
        )

    wins = (win0, win1, win2, win3)

    def init_chunk(k, _):
        neg = jnp.full((16,), -1, jnp.int32)
        for w in wins:
            w[pl.ds(k * 16, 16)] = neg
        return 0

    lax.fori_loop(0, RPAD // 16, init_chunk, 0)

    SENT = jnp.int32(1 << 20)

    def scan_one(j, win):
        idx = idx_v[pl.ds(j * 16, 16)]
        il = idx - lo
        owned = (il >= 0) & (il < nrows)
        key = jnp.where(owned, il * 16 + lane, SENT + lane)
        bvec = j * 16 + lane
        ks, vs = plsc.sort_key_val(key, bvec)
        ils = lax.shift_right_arithmetic(ks, 4)
        nxt = ils.at[jnp.minimum(lane + 1, 15)].get(mode="promise_in_bounds")
        own_s = ks < SENT
        is_last = own_s & ((lane == 15) | (ils != nxt))
        addr = jnp.where(own_s, ils, R)
        plsc.store_scatter(win, [addr], vs, mask=is_last)

    def scan_step(s, _):
        for k in range(4):
            scan_one(s * 4 + k, wins[k])
        return 0

    lax.fori_loop(0, B // 16 // 4, scan_step, 0)

    def compact_chunk(k, carry):
        off, menc = carry
        wv = jnp.maximum(
            jnp.maximum(win0[pl.ds(k * 16, 16)], win1[pl.ds(k * 16, 16)]),
            jnp.maximum(win2[pl.ds(k * 16, 16)], win3[pl.ds(k * 16, 16)]),
        )
        m = wv >= 0
        mi = m.astype(jnp.int32)
        slots = lo + k * 16 + lane
        pos = off + lax.cumsum(mi, axis=0) - 1
        pos = jnp.where(m, pos, CWROWS * 128 - 1)
        prow = lax.shift_right_arithmetic(pos, 7)
        pcol = lax.bitwise_and(pos, jnp.int32(127))
        plsc.store_scatter(cw_s, [prow, pcol], slots, mask=m)
        plsc.store_scatter(cw_b, [prow, pcol], wv, mask=m)
        enc = jnp.where(m, slots * 16384 + wv, -1)
        return off + jnp.sum(mi), jnp.maximum(menc, jnp.max(enc))

    w_t, menc = lax.fori_loop(
        0, RPAD // 16, compact_chunk, (jnp.int32(0), jnp.int32(-1))
    )

    nch = jnp.where(w_t > 0, ((w_t + 127) // 128), 0)

    @pl.when(w_t > 0)
    def _():
        s_last = lax.shift_right_arithmetic(menc, 14)
        b_last = lax.bitwise_and(menc, jnp.int32(16383))
        target = nch * 128

        def pad_chunk(k, _):
            addr = w_t + k * 16 + lane
            mk = addr < target
            addr = jnp.minimum(addr, CWROWS * 128 - 1)
            prow = lax.shift_right_arithmetic(addr, 7)
            pcol = lax.bitwise_and(addr, jnp.int32(127))
            plsc.store_scatter(cw_s, [prow, pcol], jnp.full((16,), s_last), mask=mk)
            plsc.store_scatter(cw_b, [prow, pcol], jnp.full((16,), b_last), mask=mk)
            return 0

        lax.fori_loop(0, 8, pad_chunk, 0)

    nch_v[...] = jnp.full((16,), nch, jnp.int32)
    pltpu.sync_copy(cw_s, cws_hbm.at[wid])
    pltpu.sync_copy(cw_b, cwb_hbm.at[wid])
    pltpu.sync_copy(nch_v, nch_hbm.at[wid])

    for cp in copies:
        cp.wait()
    pltpu.sync_copy(rows_v, hg_hbm.at[pl.ds(base, BPW)])


def _gru_body(x_ref, h_ref, wih_ref, whh_ref, bih_ref, bhh_ref, tbl_ref,
              out_ref, cp_ref):
    x = x_ref[...]
    h = h_ref[...]
    dn = (((1,), (1,)), ((), ()))
    gi = lax.dot_general(x, wih_ref[...], dn, preferred_element_type=jnp.float32)
    gi = gi + bih_ref[...]
    gh = lax.dot_general(h, whh_ref[...], dn, preferred_element_type=jnp.float32)
    gh = gh + bhh_ref[...]
    r = jax.nn.sigmoid(gi[:, :H] + gh[:, :H])
    z = jax.nn.sigmoid(gi[:, H:2 * H] + gh[:, H:2 * H])
    n = jnp.tanh(gi[:, 2 * H:] + r * gh[:, 2 * H:])
    out_ref[...] = (1.0 - z) * n + z * h
    cp_ref[...] = tbl_ref[...]


_TB = 512


def _gru(x, h_g, w_ih, w_hh, b_ih2, b_hh2, tbl):
    return pl.pallas_call(
        _gru_body,
        grid=(B // _TB,),
        in_specs=[
            pl.BlockSpec((_TB, I), lambda i: (i, 0)),
            pl.BlockSpec((_TB, H), lambda i: (i, 0)),
            pl.BlockSpec((3 * H, I), lambda i: (0, 0)),
            pl.BlockSpec((3 * H, H), lambda i: (0, 0)),
            pl.BlockSpec((1, 3 * H), lambda i: (0, 0)),
            pl.BlockSpec((1, 3 * H), lambda i: (0, 0)),
            pl.BlockSpec((R, H), lambda i: (i, 0)),
        ],
        out_specs=[
            pl.BlockSpec((_TB, H), lambda i: (i, 0)),
            pl.BlockSpec((R, H), lambda i: (i, 0)),
        ],
        out_shape=[
            jax.ShapeDtypeStruct((B, H), jnp.float32),
            jax.ShapeDtypeStruct((M, H), jnp.float32),
        ],
    )(x, h_g, w_ih, w_hh, b_ih2, b_hh2, tbl)


@functools.partial(
    pl.kernel,
    mesh=_mesh,
    compiler_params=_sc_params,
    scratch_types=[
        pltpu.VMEM((CWROWS, 128), jnp.int32),
        pltpu.VMEM((CWROWS, 128), jnp.int32),
        pltpu.VMEM((16,), jnp.int32),
        pltpu.VMEM((128, H), jnp.float32),
        pltpu.VMEM((128, H), jnp.float32),
        pltpu.SemaphoreType.DMA,
        pltpu.SemaphoreType.DMA,
        pltpu.SemaphoreType.DMA,
        pltpu.SemaphoreType.DMA,
    ],
)
def _scatter_update(cws_hbm, cwb_hbm, nch_hbm, hnew_hbm, out_hbm,
                    cw_s, cw_b, nch_v, rb0, rb1, g0, g1, s0, s1):
    wid = lax.axis_index("s") * NC + lax.axis_index("c")
    pltpu.sync_copy(nch_hbm.at[wid], nch_v)
    pltpu.sync_copy(cws_hbm.at[wid], cw_s)
    pltpu.sync_copy(cwb_hbm.at[wid], cw_b)
    nch = nch_v[pl.ds(0, 16)][0]

    rbufs = (rb0, rb1)
    gsems = (g0, g1)
    ssems = (s0, s1)

    def gather_chunk(c, p):
        pltpu.async_copy(hnew_hbm.at[cw_b.at[c]], rbufs[p], gsems[p])

    def wait_gather(p):
        pltpu.make_async_copy(hnew_hbm.at[cw_b.at[0]], rbufs[p], gsems[p]).wait()

    def scatter_chunk(c, p):
        pltpu.async_copy(rbufs[p], out_hbm.at[cw_s.at[c]], ssems[p])

    def wait_scatter(p):
        pltpu.make_async_copy(rbufs[p], out_hbm.at[cw_s.at[0]], ssems[p]).wait()

    @pl.when(nch > 0)
    def _():
        gather_chunk(0, 0)

    @pl.when(nch > 1)
    def _():
        gather_chunk(1, 1)

    def outer(i, _):
        for p in range(2):
            c = i * 2 + p

            @pl.when(c < nch)
            def _():
                wait_gather(p)
                scatter_chunk(c, p)
                wait_scatter(p)

                @pl.when(c + 2 < nch)
                def _():
                    gather_chunk(c + 2, p)

        return 0

    lax.fori_loop(0, (CWROWS + 1) // 2, outer, 0)


def kernel(current_time, mgn_h, delta_t, X_obs, i_obs, update, W_ih, W_hh, b_ih, b_hh):
    idx = i_obs.astype(jnp.int32)
    h_g, cws, cwb, nchs = _front(mgn_h, idx)
    h_new, tbl_copy = _gru(X_obs, h_g, W_ih, W_hh,
                           b_ih.reshape(1, 3 * H), b_hh.reshape(1, 3 * H),
                           mgn_h)
    out_ref = jax.new_ref(tbl_copy)
    _scatter_update(cws, cwb, nchs, h_new, out_ref)
    return out_ref[...]


# speedup vs baseline: 1.4508x; 1.1921x over previous
import functools

import jax
import jax.numpy as jnp
from jax import lax
from jax.experimental import pallas as pl
from jax.experimental.pallas import tpu as pltpu
from jax.experimental.pallas import tpu_sc as plsc

M = 100000
B = 16384
H = 128
I = 32

NC = 2
NS = 16
NW = NC * NS
R = 3128
BPW = B // NW
RPAD = 3136
CWROWS = 26

_mesh = plsc.VectorSubcoreMesh(core_axis_name="c", subcore_axis_name="s")
_sc_params = pltpu.CompilerParams(needs_layout_passes=False)


@functools.partial(
    pl.kernel,
    out_type=jax.ShapeDtypeStruct((B, H), jnp.float32),
    mesh=_mesh,
    compiler_params=_sc_params,
    scratch_types=[
        pltpu.VMEM((BPW,), jnp.int32),
        pltpu.VMEM((BPW, H), jnp.float32),
        pltpu.SemaphoreType.DMA,
    ],
)
def _gather_rows(table_hbm, idx_hbm, out_hbm, idx_v, rows_v, sem):
    wid = lax.axis_index("s") * NC + lax.axis_index("c")
    base = wid * BPW
    pltpu.sync_copy(idx_hbm.at[pl.ds(base, BPW)], idx_v)
    copies = []
    for c in range(BPW // 128):
        copies.append(
            pltpu.async_copy(
                table_hbm.at[idx_v.at[pl.ds(c * 128, 128)]],
                rows_v.at[pl.ds(c * 128, 128)],
                sem,
            )
        )
    for cp in copies:
        cp.wait()
    pltpu.sync_copy(rows_v, out_hbm.at[pl.ds(base, BPW)])


@functools.partial(
    pl.kernel,
    out_type=(
        jax.ShapeDtypeStruct((NW, CWROWS, 128), jnp.int32),
        jax.ShapeDtypeStruct((NW, CWROWS, 128), jnp.int32),
        jax.ShapeDtypeStruct((NW, 16), jnp.int32),
    ),
    mesh=_mesh,
    compiler_params=_sc_params,
    scratch_types=[
        pltpu.VMEM((B,), jnp.int32),
        pltpu.VMEM((RPAD,), jnp.int32),
        pltpu.VMEM((RPAD,), jnp.int32),
        pltpu.VMEM((RPAD,), jnp.int32),
        pltpu.VMEM((RPAD,), jnp.int32),
        pltpu.VMEM((CWROWS, 128), jnp.int32),
        pltpu.VMEM((CWROWS, 128), jnp.int32),
        pltpu.VMEM((16,), jnp.int32),
    ],
)
def _winner_scan(idx_hbm, cws_hbm, cwb_hbm, nch_hbm,
                 idx_v, win0, win1, win2, win3, cw_s, cw_b, nch_v):
    wid = lax.axis_index("s") * NC + lax.axis_index("c")
    lo = wid * R
    nrows = jnp.minimum(jnp.int32(R), jnp.int32(M) - lo)
    lane = lax.iota(jnp.int32, 16)

    pltpu.sync_copy(idx_hbm, idx_v)

    wins = (win0, win1, win2, win3)

    def init_chunk(k, _):
        neg = jnp.full((16,), -1, jnp.int32)
        for w in wins:
            w[pl.ds(k * 16, 16)] = neg
        return 0

    lax.fori_loop(0, RPAD // 16, init_chunk, 0)

    SENT = jnp.int32(1 << 20)

    def scan_one(j, win):
        idx = idx_v[pl.ds(j * 16, 16)]
        il = idx - lo
        owned = (il >= 0) & (il < nrows)
        key = jnp.where(owned, il * 16 + lane, SENT + lane)
        bvec = j * 16 + lane
        ks, vs = plsc.sort_key_val(key, bvec)
        ils = lax.shift_right_arithmetic(ks, 4)
        nxt = ils.at[jnp.minimum(lane + 1, 15)].get(mode="promise_in_bounds")
        own_s = ks < SENT
        is_last = own_s & ((lane == 15) | (ils != nxt))
        addr = jnp.where(own_s, ils, R)
        plsc.store_scatter(win, [addr], vs, mask=is_last)

    def scan_step(s, _):
        for k in range(4):
            scan_one(s * 4 + k, wins[k])
        return 0

    lax.fori_loop(0, B // 16 // 4, scan_step, 0)

    def compact_chunk(k, carry):
        off, menc = carry
        wv = jnp.maximum(
            jnp.maximum(win0[pl.ds(k * 16, 16)], win1[pl.ds(k * 16, 16)]),
            jnp.maximum(win2[pl.ds(k * 16, 16)], win3[pl.ds(k * 16, 16)]),
        )
        m = wv >= 0
        mi = m.astype(jnp.int32)
        slots = lo + k * 16 + lane
        pos = off + lax.cumsum(mi, axis=0) - 1
        pos = jnp.where(m, pos, CWROWS * 128 - 1)
        prow = lax.shift_right_arithmetic(pos, 7)
        pcol = lax.bitwise_and(pos, jnp.int32(127))
        plsc.store_scatter(cw_s, [prow, pcol], slots, mask=m)
        plsc.store_scatter(cw_b, [prow, pcol], wv, mask=m)
        enc = jnp.where(m, slots * 16384 + wv, -1)
        return off + jnp.sum(mi), jnp.maximum(menc, jnp.max(enc))

    w_t, menc = lax.fori_loop(
        0, RPAD // 16, compact_chunk, (jnp.int32(0), jnp.int32(-1))
    )

    nch = jnp.where(w_t > 0, ((w_t + 127) // 128), 0)

    @pl.when(w_t > 0)
    def _():
        s_last = lax.shift_right_arithmetic(menc, 14)
        b_last = lax.bitwise_and(menc, jnp.int32(16383))
        target = nch * 128

        def pad_chunk(k, _):
            addr = w_t + k * 16 + lane
            mk = addr < target
            addr = jnp.minimum(addr, CWROWS * 128 - 1)
            prow = lax.shift_right_arithmetic(addr, 7)
            pcol = lax.bitwise_and(addr, jnp.int32(127))
            plsc.store_scatter(cw_s, [prow, pcol], jnp.full((16,), s_last), mask=mk)
            plsc.store_scatter(cw_b, [prow, pcol], jnp.full((16,), b_last), mask=mk)
            return 0

        lax.fori_loop(0, 8, pad_chunk, 0)

    nch_v[...] = jnp.full((16,), nch, jnp.int32)
    pltpu.sync_copy(cw_s, cws_hbm.at[wid])
    pltpu.sync_copy(cw_b, cwb_hbm.at[wid])
    pltpu.sync_copy(nch_v, nch_hbm.at[wid])


def _gru_body(x_ref, h_ref, wih_ref, whh_ref, bih_ref, bhh_ref, tbl_ref,
              out_ref, cp_ref):
    x = x_ref[...]
    h = h_ref[...]
    dn = (((1,), (1,)), ((), ()))
    gi = lax.dot_general(x, wih_ref[...], dn, preferred_element_type=jnp.float32)
    gi = gi + bih_ref[...]
    gh = lax.dot_general(h, whh_ref[...], dn, preferred_element_type=jnp.float32)
    gh = gh + bhh_ref[...]
    r = jax.nn.sigmoid(gi[:, :H] + gh[:, :H])
    z = jax.nn.sigmoid(gi[:, H:2 * H] + gh[:, H:2 * H])
    n = jnp.tanh(gi[:, 2 * H:] + r * gh[:, 2 * H:])
    out_ref[...] = (1.0 - z) * n + z * h
    cp_ref[...] = tbl_ref[...]


_TB = 512


def _gru(x, h_g, w_ih, w_hh, b_ih2, b_hh2, tbl):
    return pl.pallas_call(
        _gru_body,
        grid=(B // _TB,),
        in_specs=[
            pl.BlockSpec((_TB, I), lambda i: (i, 0)),
            pl.BlockSpec((_TB, H), lambda i: (i, 0)),
            pl.BlockSpec((3 * H, I), lambda i: (0, 0)),
            pl.BlockSpec((3 * H, H), lambda i: (0, 0)),
            pl.BlockSpec((1, 3 * H), lambda i: (0, 0)),
            pl.BlockSpec((1, 3 * H), lambda i: (0, 0)),
            pl.BlockSpec((R, H), lambda i: (i, 0)),
        ],
        out_specs=[
            pl.BlockSpec((_TB, H), lambda i: (i, 0)),
            pl.BlockSpec((R, H), lambda i: (i, 0)),
        ],
        out_shape=[
            jax.ShapeDtypeStruct((B, H), jnp.float32),
            jax.ShapeDtypeStruct((M, H), jnp.float32),
        ],
    )(x, h_g, w_ih, w_hh, b_ih2, b_hh2, tbl)


@functools.partial(
    pl.kernel,
    mesh=_mesh,
    compiler_params=_sc_params,
    scratch_types=[
        pltpu.VMEM((CWROWS, 128), jnp.int32),
        pltpu.VMEM((CWROWS, 128), jnp.int32),
        pltpu.VMEM((16,), jnp.int32),
        pltpu.VMEM((128, H), jnp.float32),
        pltpu.VMEM((128, H), jnp.float32),
        pltpu.SemaphoreType.DMA,
        pltpu.SemaphoreType.DMA,
        pltpu.SemaphoreType.DMA,
        pltpu.SemaphoreType.DMA,
    ],
)
def _scatter_update(cws_hbm, cwb_hbm, nch_hbm, hnew_hbm, out_hbm,
                    cw_s, cw_b, nch_v, rb0, rb1, g0, g1, s0, s1):
    wid = lax.axis_index("s") * NC + lax.axis_index("c")
    pltpu.sync_copy(nch_hbm.at[wid], nch_v)
    pltpu.sync_copy(cws_hbm.at[wid], cw_s)
    pltpu.sync_copy(cwb_hbm.at[wid], cw_b)
    nch = nch_v[pl.ds(0, 16)][0]

    rbufs = (rb0, rb1)
    gsems = (g0, g1)
    ssems = (s0, s1)

    def gather_chunk(c, p):
        pltpu.async_copy(hnew_hbm.at[cw_b.at[c]], rbufs[p], gsems[p])

    def wait_gather(p):
        pltpu.make_async_copy(hnew_hbm.at[cw_b.at[0]], rbufs[p], gsems[p]).wait()

    def scatter_chunk(c, p):
        pltpu.async_copy(rbufs[p], out_hbm.at[cw_s.at[c]], ssems[p])

    def wait_scatter(p):
        pltpu.make_async_copy(rbufs[p], out_hbm.at[cw_s.at[0]], ssems[p]).wait()

    @pl.when(nch > 0)
    def _():
        gather_chunk(0, 0)

    @pl.when(nch > 1)
    def _():
        gather_chunk(1, 1)

    def outer(i, _):
        for p in range(2):
            c = i * 2 + p

            @pl.when(c < nch)
            def _():
                wait_gather(p)
                scatter_chunk(c, p)
                wait_scatter(p)

                @pl.when(c + 2 < nch)
                def _():
                    gather_chunk(c + 2, p)

        return 0

    lax.fori_loop(0, (CWROWS + 1) // 2, outer, 0)


def kernel(current_time, mgn_h, delta_t, X_obs, i_obs, update, W_ih, W_hh, b_ih, b_hh):
    idx = i_obs.astype(jnp.int32)
    h_g = _gather_rows(mgn_h, idx)
    cws, cwb, nchs = _winner_scan(idx)
    h_new, tbl_copy = _gru(X_obs, h_g, W_ih, W_hh,
                           b_ih.reshape(1, 3 * H), b_hh.reshape(1, 3 * H),
                           mgn_h)
    out_ref = jax.new_ref(tbl_copy)
    _scatter_update(cws, cwb, nchs, h_new, out_ref)
    return out_ref[...]


# speedup vs baseline: 1.5495x; 1.0681x over previous
import functools

import jax
import jax.numpy as jnp
from jax import lax
from jax.experimental import pallas as pl
from jax.experimental.pallas import tpu as pltpu
from jax.experimental.pallas import tpu_sc as plsc

M = 100000
B = 16384
H = 128
I = 32

NC = 2
NS = 16
NW = NC * NS
R = 3128
BPW = B // NW
RPAD = 3136
CWROWS = 26

_mesh = plsc.VectorSubcoreMesh(core_axis_name="c", subcore_axis_name="s")
_sc_params = pltpu.CompilerParams(needs_layout_passes=False)


@functools.partial(
    pl.kernel,
    out_type=jax.ShapeDtypeStruct((B, H), jnp.float32),
    mesh=_mesh,
    compiler_params=_sc_params,
    scratch_types=[
        pltpu.VMEM((BPW,), jnp.int32),
        pltpu.VMEM((BPW, H), jnp.float32),
        pltpu.SemaphoreType.DMA,
    ],
)
def _gather_rows(table_hbm, idx_hbm, out_hbm, idx_v, rows_v, sem):
    wid = lax.axis_index("s") * NC + lax.axis_index("c")
    base = wid * BPW
    pltpu.sync_copy(idx_hbm.at[pl.ds(base, BPW)], idx_v)
    copies = []
    for c in range(BPW // 128):
        copies.append(
            pltpu.async_copy(
                table_hbm.at[idx_v.at[pl.ds(c * 128, 128)]],
                rows_v.at[pl.ds(c * 128, 128)],
                sem,
            )
        )
    for cp in copies:
        cp.wait()
    pltpu.sync_copy(rows_v, out_hbm.at[pl.ds(base, BPW)])


@functools.partial(
    pl.kernel,
    out_type=(
        jax.ShapeDtypeStruct((NW, CWROWS, 128), jnp.int32),
        jax.ShapeDtypeStruct((NW, CWROWS, 128), jnp.int32),
        jax.ShapeDtypeStruct((NW, 16), jnp.int32),
    ),
    mesh=_mesh,
    compiler_params=_sc_params,
    scratch_types=[
        pltpu.VMEM((B,), jnp.int32),
        pltpu.VMEM((RPAD,), jnp.int32),
        pltpu.VMEM((RPAD,), jnp.int32),
        pltpu.VMEM((RPAD,), jnp.int32),
        pltpu.VMEM((RPAD,), jnp.int32),
        pltpu.VMEM((CWROWS, 128), jnp.int32),
        pltpu.VMEM((CWROWS, 128), jnp.int32),
        pltpu.VMEM((16,), jnp.int32),
    ],
)
def _winner_scan(idx_hbm, cws_hbm, cwb_hbm, nch_hbm,
                 idx_v, win0, win1, win2, win3, cw_s, cw_b, nch_v):
    wid = lax.axis_index("s") * NC + lax.axis_index("c")
    lo = wid * R
    nrows = jnp.minimum(jnp.int32(R), jnp.int32(M) - lo)
    lane = lax.iota(jnp.int32, 16)

    pltpu.sync_copy(idx_hbm, idx_v)

    wins = (win0, win1, win2, win3)

    def init_chunk(k, _):
        neg = jnp.full((16,), -1, jnp.int32)
        for w in wins:
            w[pl.ds(k * 16, 16)] = neg
        return 0

    lax.fori_loop(0, RPAD // 16, init_chunk, 0)

    SENT = jnp.int32(1 << 20)

    def scan_one(j, win):
        idx = idx_v[pl.ds(j * 16, 16)]
        il = idx - lo
        owned = (il >= 0) & (il < nrows)
        key = jnp.where(owned, il * 16 + lane, SENT + lane)
        bvec = j * 16 + lane
        ks, vs = plsc.sort_key_val(key, bvec)
        ils = lax.shift_right_arithmetic(ks, 4)
        nxt = ils.at[jnp.minimum(lane + 1, 15)].get(mode="promise_in_bounds")
        own_s = ks < SENT
        is_last = own_s & ((lane == 15) | (ils != nxt))
        addr = jnp.where(own_s, ils, R)
        plsc.store_scatter(win, [addr], vs, mask=is_last)

    def scan_step(s, _):
        for k in range(4):
            scan_one(s * 4 + k, wins[k])
        return 0

    lax.fori_loop(0, B // 16 // 4, scan_step, 0)

    def compact_chunk(k, carry):
        off, menc = carry
        wv = jnp.maximum(
            jnp.maximum(win0[pl.ds(k * 16, 16)], win1[pl.ds(k * 16, 16)]),
            jnp.maximum(win2[pl.ds(k * 16, 16)], win3[pl.ds(k * 16, 16)]),
        )
        m = wv >= 0
        mi = m.astype(jnp.int32)
        slots = lo + k * 16 + lane
        pos = off + lax.cumsum(mi, axis=0) - 1
        pos = jnp.where(m, pos, CWROWS * 128 - 1)
        prow = lax.shift_right_arithmetic(pos, 7)
        pcol = lax.bitwise_and(pos, jnp.int32(127))
        plsc.store_scatter(cw_s, [prow, pcol], slots, mask=m)
        plsc.store_scatter(cw_b, [prow, pcol], wv, mask=m)
        enc = jnp.where(m, slots * 16384 + wv, -1)
        return off + jnp.sum(mi), jnp.maximum(menc, jnp.max(enc))

    w_t, menc = lax.fori_loop(
        0, RPAD // 16, compact_chunk, (jnp.int32(0), jnp.int32(-1))
    )

    nch = jnp.where(w_t > 0, ((w_t + 127) // 128), 0)

    @pl.when(w_t > 0)
    def _():
        s_last = lax.shift_right_arithmetic(menc, 14)
        b_last = lax.bitwise_and(menc, jnp.int32(16383))
        target = nch * 128

        def pad_chunk(k, _):
            addr = w_t + k * 16 + lane
            mk = addr < target
            addr = jnp.minimum(addr, CWROWS * 128 - 1)
            prow = lax.shift_right_arithmetic(addr, 7)
            pcol = lax.bitwise_and(addr, jnp.int32(127))
            plsc.store_scatter(cw_s, [prow, pcol], jnp.full((16,), s_last), mask=mk)
            plsc.store_scatter(cw_b, [prow, pcol], jnp.full((16,), b_last), mask=mk)
            return 0

        lax.fori_loop(0, 8, pad_chunk, 0)

    nch_v[...] = jnp.full((16,), nch, jnp.int32)
    pltpu.sync_copy(cw_s, cws_hbm.at[wid])
    pltpu.sync_copy(cw_b, cwb_hbm.at[wid])
    pltpu.sync_copy(nch_v, nch_hbm.at[wid])


def _gru_body(x_ref, h_ref, wih_ref, whh_ref, bih_ref, bhh_ref, tbl_ref,
              out_ref, cp_ref):
    x = x_ref[...]
    h = h_ref[...]
    dn = (((1,), (1,)), ((), ()))
    gi = lax.dot_general(x, wih_ref[...], dn, preferred_element_type=jnp.float32)
    gi = gi + bih_ref[...]
    gh = lax.dot_general(h, whh_ref[...], dn, preferred_element_type=jnp.float32)
    gh = gh + bhh_ref[...]
    r = jax.nn.sigmoid(gi[:, :H] + gh[:, :H])
    z = jax.nn.sigmoid(gi[:, H:2 * H] + gh[:, H:2 * H])
    n = jnp.tanh(gi[:, 2 * H:] + r * gh[:, 2 * H:])
    out_ref[...] = (1.0 - z) * n + z * h
    cp_ref[...] = tbl_ref[...]


_TB = 1024
CPB = 6256


def _gru(x, h_g, w_ih, w_hh, b_ih2, b_hh2, tbl):
    return pl.pallas_call(
        _gru_body,
        grid=(B // _TB,),
        in_specs=[
            pl.BlockSpec((_TB, I), lambda i: (i, 0)),
            pl.BlockSpec((_TB, H), lambda i: (i, 0)),
            pl.BlockSpec((3 * H, I), lambda i: (0, 0)),
            pl.BlockSpec((3 * H, H), lambda i: (0, 0)),
            pl.BlockSpec((1, 3 * H), lambda i: (0, 0)),
            pl.BlockSpec((1, 3 * H), lambda i: (0, 0)),
            pl.BlockSpec((CPB, H), lambda i: (i, 0)),
        ],
        out_specs=[
            pl.BlockSpec((_TB, H), lambda i: (i, 0)),
            pl.BlockSpec((CPB, H), lambda i: (i, 0)),
        ],
        out_shape=[
            jax.ShapeDtypeStruct((B, H), jnp.float32),
            jax.ShapeDtypeStruct((M, H), jnp.float32),
        ],
    )(x, h_g, w_ih, w_hh, b_ih2, b_hh2, tbl)


@functools.partial(
    pl.kernel,
    mesh=_mesh,
    compiler_params=_sc_params,
    scratch_types=[
        pltpu.VMEM((CWROWS, 128), jnp.int32),
        pltpu.VMEM((CWROWS, 128), jnp.int32),
        pltpu.VMEM((16,), jnp.int32),
        pltpu.VMEM((128, H), jnp.float32),
        pltpu.VMEM((128, H), jnp.float32),
        pltpu.SemaphoreType.DMA,
        pltpu.SemaphoreType.DMA,
        pltpu.SemaphoreType.DMA,
        pltpu.SemaphoreType.DMA,
    ],
)
def _scatter_update(cws_hbm, cwb_hbm, nch_hbm, hnew_hbm, out_hbm,
                    cw_s, cw_b, nch_v, rb0, rb1, g0, g1, s0, s1):
    wid = lax.axis_index("s") * NC + lax.axis_index("c")
    pltpu.sync_copy(nch_hbm.at[wid], nch_v)
    pltpu.sync_copy(cws_hbm.at[wid], cw_s)
    pltpu.sync_copy(cwb_hbm.at[wid], cw_b)
    nch = nch_v[pl.ds(0, 16)][0]

    rbufs = (rb0, rb1)
    gsems = (g0, g1)
    ssems = (s0, s1)

    def gather_chunk(c, p):
        pltpu.async_copy(hnew_hbm.at[cw_b.at[c]], rbufs[p], gsems[p])

    def wait_gather(p):
        pltpu.make_async_copy(hnew_hbm.at[cw_b.at[0]], rbufs[p], gsems[p]).wait()

    def scatter_chunk(c, p):
        pltpu.async_copy(rbufs[p], out_hbm.at[cw_s.at[c]], ssems[p])

    def wait_scatter(p):
        pltpu.make_async_copy(rbufs[p], out_hbm.at[cw_s.at[0]], ssems[p]).wait()

    @pl.when(nch > 0)
    def _():
        gather_chunk(0, 0)

    @pl.when(nch > 1)
    def _():
        gather_chunk(1, 1)

    def outer(i, _):
        for p in range(2):
            c = i * 2 + p

            @pl.when(c < nch)
            def _():
                wait_gather(p)
                scatter_chunk(c, p)
                wait_scatter(p)

                @pl.when(c + 2 < nch)
                def _():
                    gather_chunk(c + 2, p)

        return 0

    lax.fori_loop(0, (CWROWS + 1) // 2, outer, 0)


def kernel(current_time, mgn_h, delta_t, X_obs, i_obs, update, W_ih, W_hh, b_ih, b_hh):
    idx = i_obs.astype(jnp.int32)
    h_g = _gather_rows(mgn_h, idx)
    cws, cwb, nchs = _winner_scan(idx)
    h_new, tbl_copy = _gru(X_obs, h_g, W_ih, W_hh,
                           b_ih.reshape(1, 3 * H), b_hh.reshape(1, 3 * H),
                           mgn_h)
    out_ref = jax.new_ref(tbl_copy)
    _scatter_update(cws, cwb, nchs, h_new, out_ref)
    return out_ref[...]
